# baseline (speedup 1.0000x reference)
import functools

import jax
import jax.numpy as jnp
from jax import lax
from jax.experimental import pallas as pl
from jax.experimental.pallas import tpu as pltpu
from jax.experimental.pallas import tpu_sc as plsc

_POS_OFFSET = 2


def kernel(input, weights):
    b, t = input.shape
    d = weights.shape[1]

    NC, NS = 2, 16
    NW = NC * NS
    rows_per_w = t // NW
    CHUNK = 64
    n_chunks = rows_per_w // CHUNK

    mesh = plsc.VectorSubcoreMesh(core_axis_name="c", subcore_axis_name="s")

    @functools.partial(
        pl.kernel,
        mesh=mesh,
        out_type=jax.ShapeDtypeStruct((b * t * d,), weights.dtype),
        scratch_types=[
            pltpu.VMEM((CHUNK * d,), weights.dtype),
            pltpu.VMEM((CHUNK * d,), weights.dtype),
            pltpu.SemaphoreType.DMA,
            pltpu.SemaphoreType.DMA,
            pltpu.SemaphoreType.DMA,
            pltpu.SemaphoreType.DMA,
        ],
    )
    def _posemb(w_hbm, out_hbm, buf_a, buf_b, sin_a, sin_b, sout_a, sout_b):
        wid = lax.axis_index("s") * NC + lax.axis_index("c")
        base = wid * rows_per_w
        bufs, sins, souts = [buf_a, buf_b], [sin_a, sin_b], [sout_a, sout_b]

        def start_in(ci):
            r0 = base + ci * CHUNK
            return pltpu.async_copy(
                w_hbm.at[pl.ds((_POS_OFFSET + r0) * d, CHUNK * d)],
                bufs[ci % 2], sins[ci % 2])

        def start_outs(ci):
            r0 = base + ci * CHUNK
            return [
                pltpu.async_copy(
                    bufs[ci % 2],
                    out_hbm.at[pl.ds((bi * t + r0) * d, CHUNK * d)],
                    souts[ci % 2])
                for bi in range(b)
            ]

        ins, outs = {}, {}
        for ci in range(min(2, n_chunks)):
            ins[ci] = start_in(ci)
        for ci in range(min(2, n_chunks)):
            ins[ci].wait()
            outs[ci] = start_outs(ci)
        for ci in range(2, n_chunks):
            for cp in outs[ci - 2]:
                cp.wait()
            ins[ci] = start_in(ci)
            ins[ci].wait()
            outs[ci] = start_outs(ci)
        for ci in range(max(0, n_chunks - 2), n_chunks):
            for cp in outs[ci]:
                cp.wait()

    return _posemb(weights.reshape(-1)).reshape(b, t, d)


# speedup vs baseline: 1.0028x; 1.0028x over previous
import functools

import jax
import jax.numpy as jnp
from jax import lax
from jax.experimental import pallas as pl
from jax.experimental.pallas import tpu as pltpu
from jax.experimental.pallas import tpu_sc as plsc

_POS_OFFSET = 2


def kernel(input, weights):
    b, t = input.shape
    d = weights.shape[1]

    NC, NS = 2, 16
    NW = NC * NS
    rows_per_w = t // NW
    CHUNK = 64
    n_chunks = rows_per_w // CHUNK
    mesh = plsc.VectorSubcoreMesh(core_axis_name="c", subcore_axis_name="s")

    @functools.partial(
        pl.kernel,
        mesh=mesh,
        out_type=jax.ShapeDtypeStruct((b, t, d), weights.dtype),
        scratch_types=[
            pltpu.VMEM((CHUNK, d), weights.dtype),
            pltpu.VMEM((CHUNK, d), weights.dtype),
            pltpu.SemaphoreType.DMA,
            pltpu.SemaphoreType.DMA,
            pltpu.SemaphoreType.DMA,
            pltpu.SemaphoreType.DMA,
        ],
        compiler_params=pltpu.CompilerParams(use_tc_tiling_on_sc=False),
    )
    def _posemb(w_hbm, out_hbm, buf_a, buf_b, sin_a, sin_b, sout_a, sout_b):
        wid = lax.axis_index("s") * NC + lax.axis_index("c")
        base = wid * rows_per_w
        bufs, sins, souts = [buf_a, buf_b], [sin_a, sin_b], [sout_a, sout_b]

        def start_in(ci):
            r0 = base + ci * CHUNK
            return pltpu.async_copy(
                w_hbm.at[pl.ds(_POS_OFFSET + r0, CHUNK)],
                bufs[ci % 2], sins[ci % 2])

        def start_outs(ci):
            r0 = base + ci * CHUNK
            return [
                pltpu.async_copy(
                    bufs[ci % 2],
                    out_hbm.at[bi, pl.ds(r0, CHUNK)],
                    souts[ci % 2])
                for bi in range(b)
            ]

        ins, outs = {}, {}
        for ci in range(min(2, n_chunks)):
            ins[ci] = start_in(ci)
        for ci in range(min(2, n_chunks)):
            ins[ci].wait()
            outs[ci] = start_outs(ci)
        for ci in range(2, n_chunks):
            for cp in outs[ci - 2]:
                cp.wait()
            ins[ci] = start_in(ci)
            ins[ci].wait()
            outs[ci] = start_outs(ci)
        for ci in range(max(0, n_chunks - 2), n_chunks):
            for cp in outs[ci]:
                cp.wait()

    return _posemb(weights)


# speedup vs baseline: 3.0549x; 3.0462x over previous
import functools

import jax
import jax.numpy as jnp
from jax import lax
from jax.experimental import pallas as pl
from jax.experimental.pallas import tpu as pltpu
from jax.experimental.pallas import tpu_sc as plsc

_POS_OFFSET = 2


def kernel(input, weights):
    b, t = input.shape
    d = weights.shape[1]

    NC, NS = 2, 16
    NW = NC * NS
    rows_per_w = t // NW
    CHUNK = 64
    n_chunks = rows_per_w // CHUNK
    L = 16
    mesh = plsc.VectorSubcoreMesh(core_axis_name="c", subcore_axis_name="s")

    @functools.partial(
        pl.kernel,
        mesh=mesh,
        out_type=jax.ShapeDtypeStruct((b, t, d), weights.dtype),
        scratch_types=[
            pltpu.VMEM((CHUNK, d), weights.dtype),
            pltpu.VMEM((CHUNK, d), weights.dtype),
            [pltpu.VMEM((CHUNK,), jnp.int32) for _ in range(n_chunks)],
            pltpu.SemaphoreType.DMA,
            pltpu.SemaphoreType.DMA,
            pltpu.SemaphoreType.DMA,
            pltpu.SemaphoreType.DMA,
        ],
    )
    def _posemb(w_hbm, out_hbm, buf_a, buf_b, idxs, sin_a, sin_b, sout_a, sout_b):
        wid = lax.axis_index("s") * NC + lax.axis_index("c")
        base = wid * rows_per_w
        bufs, sins, souts = [buf_a, buf_b], [sin_a, sin_b], [sout_a, sout_b]

        lane = lax.iota(jnp.int32, L)
        for ci in range(n_chunks):
            for j in range(CHUNK // L):
                idxs[ci][pl.ds(j * L, L)] = (
                    _POS_OFFSET + base + ci * CHUNK + j * L) + lane

        def start_in(ci):
            return pltpu.async_copy(
                w_hbm.at[idxs[ci]], bufs[ci % 2], sins[ci % 2])

        def start_outs(ci):
            r0 = base + ci * CHUNK
            return [
                pltpu.async_copy(
                    bufs[ci % 2],
                    out_hbm.at[bi, pl.ds(r0, CHUNK)],
                    souts[ci % 2])
                for bi in range(b)
            ]

        ins, outs = {}, {}
        for ci in range(min(2, n_chunks)):
            ins[ci] = start_in(ci)
        for ci in range(min(2, n_chunks)):
            ins[ci].wait()
            outs[ci] = start_outs(ci)
        for ci in range(2, n_chunks):
            for cp in outs[ci - 2]:
                cp.wait()
            ins[ci] = start_in(ci)
            ins[ci].wait()
            outs[ci] = start_outs(ci)
        for ci in range(max(0, n_chunks - 2), n_chunks):
            for cp in outs[ci]:
                cp.wait()

    return _posemb(weights)
